# baseline (speedup 1.0000x reference)
import jax
import jax.numpy as jnp
from jax import lax
from jax.experimental import pallas as pl
from jax.experimental.pallas import tpu as pltpu
from jax.experimental.pallas import tpu_sc as plsc

NC = 2
NS = 16
NW = NC * NS

CHG = 32
NBUF = 4
GLA = 2
RPB = 4
CHUNKS = (16, 32, 32, 32, 16)
LN_EPS = 1e-12


def _sc_gather(ids, tok_emb):
    tokens, = ids.shape
    vocab, d_model = tok_emb.shape
    assert tokens % NW == 0
    tpw = tokens // NW
    assert tpw % CHG == 0
    nch = tpw // CHG

    mesh = plsc.VectorSubcoreMesh(
        core_axis_name="c", subcore_axis_name="s", num_cores=NC, num_subcores=NS
    )

    def body(ids_h, tok_h, out_h, idx_all, rows_v, gsem, osem):
        w = lax.axis_index("s") * NC + lax.axis_index("c")
        wbase = w * tpw
        pltpu.sync_copy(ids_h.at[pl.ds(wbase, tpw)], idx_all)
        pend_g = {}
        pend_o = {}
        for c in range(nch + GLA):
            if c - NBUF in pend_o:
                pend_o.pop(c - NBUF).wait()
            if c < nch:
                pend_g[c] = pltpu.async_copy(
                    tok_h.at[idx_all.at[pl.ds(c * CHG, CHG)]],
                    rows_v.at[c % NBUF], gsem,
                )
            cg = c - GLA
            if cg in pend_g:
                pend_g.pop(cg).wait()
                pend_o[cg] = pltpu.async_copy(
                    rows_v.at[cg % NBUF],
                    out_h.at[pl.ds(wbase + cg * CHG, CHG)],
                    osem,
                )
        for c in sorted(pend_o):
            pend_o[c].wait()

    return pl.kernel(
        body,
        out_type=jax.ShapeDtypeStruct((tokens, d_model), jnp.float32),
        mesh=mesh,
        compiler_params=pltpu.CompilerParams(needs_layout_passes=False),
        scratch_types=[
            pltpu.VMEM((tpw,), jnp.int32),
            pltpu.VMEM((NBUF, CHG, d_model), jnp.float32),
            pltpu.SemaphoreType.DMA,
            pltpu.SemaphoreType.DMA,
        ],
    )(ids, tok_emb)


def _tc_addnorm(prev_out, gat_k, segf, pos_emb, seg_emb, gamma, beta,
                batch, base_rows):
    rows_k, seq, d_model = gat_k.shape

    def body(*refs):
        gat_ref, segf_ref, pos_ref, sege_ref, gam_ref, bet_ref = refs[-7:-1]
        out_ref = refs[-1]
        base = pos_ref[...] + sege_ref[0, :][None, :]
        diff = (sege_ref[1, :] - sege_ref[0, :])[None, None, :]
        emb = gat_ref[...] + base[None, :, :] + segf_ref[...] * diff
        mean = jnp.mean(emb, axis=-1, keepdims=True)
        cent = emb - mean
        var = jnp.mean(cent * cent, axis=-1, keepdims=True)
        rstd = lax.rsqrt(var + LN_EPS)
        out_ref[...] = (cent * rstd) * gam_ref[0, :][None, None, :] \
            + bet_ref[0, :][None, None, :]

    base_blk = base_rows // RPB
    in_specs = [
        pl.BlockSpec((RPB, seq, d_model), lambda i: (i, 0, 0)),
        pl.BlockSpec((RPB, seq, 1), lambda i: (i + base_blk, 0, 0)),
        pl.BlockSpec((seq, d_model), lambda i: (0, 0)),
        pl.BlockSpec((2, d_model), lambda i: (0, 0)),
        pl.BlockSpec((1, d_model), lambda i: (0, 0)),
        pl.BlockSpec((1, d_model), lambda i: (0, 0)),
    ]
    args = (gat_k, segf, pos_emb, seg_emb,
            gamma.reshape(1, -1), beta.reshape(1, -1))
    aliases = {}
    if prev_out is not None:
        in_specs = [pl.BlockSpec(memory_space=pl.ANY)] + in_specs
        args = (prev_out,) + args
        aliases = {0: 0}

    return pl.pallas_call(
        body,
        grid=(rows_k // RPB,),
        in_specs=in_specs,
        out_specs=pl.BlockSpec((RPB, seq, d_model),
                               lambda i: (i + base_blk, 0, 0)),
        out_shape=jax.ShapeDtypeStruct((batch, seq, d_model), jnp.float32),
        input_output_aliases=aliases,
    )(*args)


def kernel(input_ids, segment_ids, tok_emb, pos_emb, seg_emb, gamma, beta):
    batch, seq = input_ids.shape
    _, d_model = tok_emb.shape
    assert sum(CHUNKS) == batch and all(bk % RPB == 0 for bk in CHUNKS)
    ids = input_ids.reshape(-1).astype(jnp.int32)
    segf = segment_ids.astype(jnp.float32).reshape(batch, seq, 1)

    bases = [sum(CHUNKS[:k]) for k in range(len(CHUNKS))]
    gats = [
        _sc_gather(ids[b * seq:(b + bk) * seq], tok_emb)
        .reshape(bk, seq, d_model)
        for b, bk in zip(bases, CHUNKS)
    ]
    out = None
    for g, b in zip(gats, bases):
        out = _tc_addnorm(out, g, segf, pos_emb, seg_emb, gamma, beta,
                          batch, b)
    return out


# speedup vs baseline: 1.0721x; 1.0721x over previous
import jax
import jax.numpy as jnp
from jax import lax
from jax.experimental import pallas as pl
from jax.experimental.pallas import tpu as pltpu
from jax.experimental.pallas import tpu_sc as plsc

NC = 2
NS = 16
NW = NC * NS

CHG = 32
NBUF = 4
GLA = 2
RPB = 4
CHUNKS = (32, 32, 32, 32)
LN_EPS = 1e-12


def _sc_gather(ids, tok_emb):
    tokens, = ids.shape
    vocab, d_model = tok_emb.shape
    assert tokens % NW == 0
    tpw = tokens // NW
    assert tpw % CHG == 0
    nch = tpw // CHG

    mesh = plsc.VectorSubcoreMesh(
        core_axis_name="c", subcore_axis_name="s", num_cores=NC, num_subcores=NS
    )

    def body(ids_h, tok_h, out_h, idx_all, rows_v, gsem, osem):
        w = lax.axis_index("s") * NC + lax.axis_index("c")
        wbase = w * tpw
        pltpu.sync_copy(ids_h.at[pl.ds(wbase, tpw)], idx_all)
        pend_g = {}
        pend_o = {}
        for c in range(nch + GLA):
            if c - NBUF in pend_o:
                pend_o.pop(c - NBUF).wait()
            if c < nch:
                pend_g[c] = pltpu.async_copy(
                    tok_h.at[idx_all.at[pl.ds(c * CHG, CHG)]],
                    rows_v.at[c % NBUF], gsem,
                )
            cg = c - GLA
            if cg in pend_g:
                pend_g.pop(cg).wait()
                pend_o[cg] = pltpu.async_copy(
                    rows_v.at[cg % NBUF],
                    out_h.at[pl.ds(wbase + cg * CHG, CHG)],
                    osem,
                )
        for c in sorted(pend_o):
            pend_o[c].wait()

    return pl.kernel(
        body,
        out_type=jax.ShapeDtypeStruct((tokens, d_model), jnp.float32),
        mesh=mesh,
        compiler_params=pltpu.CompilerParams(needs_layout_passes=False),
        scratch_types=[
            pltpu.VMEM((tpw,), jnp.int32),
            pltpu.VMEM((NBUF, CHG, d_model), jnp.float32),
            pltpu.SemaphoreType.DMA,
            pltpu.SemaphoreType.DMA,
        ],
    )(ids, tok_emb)


def _tc_addnorm(prev_out, gat_k, segf, pos_emb, seg_emb, gamma, beta,
                batch, base_rows):
    rows_k, seq, d_model = gat_k.shape

    def body(*refs):
        gat_ref, segf_ref, pos_ref, sege_ref, gam_ref, bet_ref = refs[-7:-1]
        out_ref = refs[-1]
        base = pos_ref[...] + sege_ref[0, :][None, :]
        diff = (sege_ref[1, :] - sege_ref[0, :])[None, None, :]
        s3 = segf_ref[0][:, :, None]
        emb = gat_ref[...] + base[None, :, :] + s3 * diff
        mean = jnp.mean(emb, axis=-1, keepdims=True)
        cent = emb - mean
        var = jnp.mean(cent * cent, axis=-1, keepdims=True)
        rstd = lax.rsqrt(var + LN_EPS)
        out_ref[...] = (cent * rstd) * gam_ref[0, :][None, None, :] \
            + bet_ref[0, :][None, None, :]

    base_blk = base_rows // RPB
    in_specs = [
        pl.BlockSpec((RPB, seq, d_model), lambda i: (i, 0, 0)),
        pl.BlockSpec((1, RPB, seq), lambda i: (i + base_blk, 0, 0)),
        pl.BlockSpec((seq, d_model), lambda i: (0, 0)),
        pl.BlockSpec((2, d_model), lambda i: (0, 0)),
        pl.BlockSpec((1, d_model), lambda i: (0, 0)),
        pl.BlockSpec((1, d_model), lambda i: (0, 0)),
    ]
    args = (gat_k, segf, pos_emb, seg_emb,
            gamma.reshape(1, -1), beta.reshape(1, -1))
    aliases = {}
    if prev_out is not None:
        in_specs = [pl.BlockSpec(memory_space=pl.ANY)] + in_specs
        args = (prev_out,) + args
        aliases = {0: 0}

    return pl.pallas_call(
        body,
        grid=(rows_k // RPB,),
        in_specs=in_specs,
        out_specs=pl.BlockSpec((RPB, seq, d_model),
                               lambda i: (i + base_blk, 0, 0)),
        out_shape=jax.ShapeDtypeStruct((batch, seq, d_model), jnp.float32),
        input_output_aliases=aliases,
    )(*args)


def kernel(input_ids, segment_ids, tok_emb, pos_emb, seg_emb, gamma, beta):
    batch, seq = input_ids.shape
    _, d_model = tok_emb.shape
    assert sum(CHUNKS) == batch and all(bk % RPB == 0 for bk in CHUNKS)
    ids = input_ids.reshape(-1).astype(jnp.int32)
    segf = segment_ids.astype(jnp.float32).reshape(batch // RPB, RPB, seq)

    bases = [sum(CHUNKS[:k]) for k in range(len(CHUNKS))]
    gats = [
        _sc_gather(ids[b * seq:(b + bk) * seq], tok_emb)
        .reshape(bk, seq, d_model)
        for b, bk in zip(bases, CHUNKS)
    ]
    out = None
    for g, b in zip(gats, bases):
        out = _tc_addnorm(out, g, segf, pos_emb, seg_emb, gamma, beta,
                          batch, b)
    return out
